# baseline (speedup 1.0000x reference)
import functools
import math

import jax
import jax.numpy as jnp
from jax import lax
from jax.experimental import pallas as pl
from jax.experimental.pallas import tpu as pltpu
from jax.experimental.pallas import tpu_sc as plsc

D_MODEL = 64
SCALE = math.sqrt(D_MODEL)

NC = 2
NS = 16
NW = NC * NS
GMAX = 128
NBUF = 4


def _emb_body(x_hbm, lut_hbm, out_hbm, idx_v,
              gb0, gb1, gb2, gb3, sb0, sb1, sb2, sb3,
              gs0, gs1, gs2, gs3, ss0, ss1, ss2, ss3):
    gbufs = (gb0, gb1, gb2, gb3)
    sbufs = (sb0, sb1, sb2, sb3)
    gsems = (gs0, gs1, gs2, gs3)
    ssems = (ss0, ss1, ss2, ss3)

    rows_w, n_seq = idx_v.shape
    rem = n_seq - GMAX
    wid = lax.axis_index("s") * NC + lax.axis_index("c")
    row0 = wid * rows_w

    pltpu.sync_copy(x_hbm.at[pl.ds(row0, rows_w)], idx_v)

    def issue_gather(j, b):
        pltpu.async_copy(
            lut_hbm.at[idx_v.at[j, pl.ds(0, GMAX)]],
            gbufs[b].at[pl.ds(0, GMAX)], gsems[b])
        pltpu.async_copy(
            lut_hbm.at[idx_v.at[j, pl.ds(GMAX, rem)]],
            gbufs[b].at[pl.ds(GMAX, rem)], gsems[b])

    def wait_gather(j, b):
        pltpu.make_async_copy(
            lut_hbm.at[idx_v.at[j, pl.ds(0, GMAX)]],
            gbufs[b].at[pl.ds(0, GMAX)], gsems[b]).wait()
        pltpu.make_async_copy(
            lut_hbm.at[idx_v.at[j, pl.ds(GMAX, rem)]],
            gbufs[b].at[pl.ds(GMAX, rem)], gsems[b]).wait()

    for b in range(NBUF):
        issue_gather(b, b)

    def outer_body(outer, carry):
        for b in range(NBUF):
            j = outer * NBUF + b
            wait_gather(j, b)

            @pl.when(outer > 0)
            def _drain(b=b, j=j):
                pltpu.make_async_copy(
                    sbufs[b], out_hbm.at[row0 + j - NBUF], ssems[b]).wait()

            def sc_t(t, gb=gbufs[b], sb=sbufs[b]):
                for q in range(4):
                    sb[t, pl.ds(16 * q, 16)] = (
                        gb[t, pl.ds(16 * q, 16)] * SCALE)

            plsc.parallel_loop(0, n_seq, unroll=4)(sc_t)

            @pl.when(j + NBUF < rows_w)
            def _issue(b=b, j=j):
                issue_gather(j + NBUF, b)

            pltpu.async_copy(sbufs[b], out_hbm.at[row0 + j], ssems[b])
        return carry

    lax.fori_loop(0, rows_w // NBUF, outer_body, 0)

    for b in range(NBUF):
        j = rows_w - NBUF + b
        pltpu.make_async_copy(
            sbufs[b], out_hbm.at[row0 + j], ssems[b]).wait()


@jax.jit
def _emb_call(x, lut):
    nb, ns = x.shape
    mesh = plsc.VectorSubcoreMesh(core_axis_name="c", subcore_axis_name="s")
    fn = functools.partial(
        pl.kernel,
        out_type=jax.ShapeDtypeStruct((nb, ns, D_MODEL), jnp.float32),
        mesh=mesh,
        scratch_types=(
            [pltpu.VMEM((nb // NW, ns), jnp.int32)]
            + [pltpu.VMEM((ns, D_MODEL), jnp.float32)] * (2 * NBUF)
            + [pltpu.SemaphoreType.DMA] * (2 * NBUF)
        ),
        compiler_params=pltpu.CompilerParams(
            use_tc_tiling_on_sc=False, needs_layout_passes=False),
    )(_emb_body)
    return fn(x, lut)


def kernel(x, lut):
    return _emb_call(x.astype(jnp.int32), lut)


# speedup vs baseline: 1.0006x; 1.0006x over previous
import functools
import math

import jax
import jax.numpy as jnp
from jax import lax
from jax.experimental import pallas as pl
from jax.experimental.pallas import tpu as pltpu
from jax.experimental.pallas import tpu_sc as plsc

D_MODEL = 64
SCALE = math.sqrt(D_MODEL)

NC = 2
NS = 16
NW = NC * NS
GMAX = 128
NBUF = 4


def _emb_body(x_hbm, lut_hbm, out_hbm, idx_v,
              gb0, gb1, gb2, gb3, sb0, sb1, sb2, sb3,
              gs0, gs1, gs2, gs3, ss0, ss1, ss2, ss3):
    gbufs = (gb0, gb1, gb2, gb3)
    sbufs = (sb0, sb1, sb2, sb3)
    gsems = (gs0, gs1, gs2, gs3)
    ssems = (ss0, ss1, ss2, ss3)

    rows_w, n_seq = idx_v.shape
    rem = n_seq - GMAX
    wid = lax.axis_index("s") * NC + lax.axis_index("c")
    row0 = wid * rows_w

    pltpu.sync_copy(x_hbm.at[pl.ds(row0, rows_w)], idx_v)

    def issue_gather(j, b):
        pltpu.async_copy(
            lut_hbm.at[idx_v.at[j, pl.ds(0, GMAX)]],
            gbufs[b].at[pl.ds(0, GMAX)], gsems[b])
        pltpu.async_copy(
            lut_hbm.at[idx_v.at[j, pl.ds(GMAX, rem)]],
            gbufs[b].at[pl.ds(GMAX, rem)], gsems[b])

    def wait_gather(j, b):
        pltpu.make_async_copy(
            lut_hbm.at[idx_v.at[j, pl.ds(0, GMAX)]],
            gbufs[b].at[pl.ds(0, GMAX)], gsems[b]).wait()
        pltpu.make_async_copy(
            lut_hbm.at[idx_v.at[j, pl.ds(GMAX, rem)]],
            gbufs[b].at[pl.ds(GMAX, rem)], gsems[b]).wait()

    for b in range(NBUF):
        issue_gather(b, b)

    def outer_body(outer, carry):
        for b in range(NBUF):
            j = outer * NBUF + b
            wait_gather(j, b)

            @pl.when(outer > 0)
            def _drain(b=b, j=j):
                pltpu.make_async_copy(
                    sbufs[b],
                    out_hbm.at[
                        pl.ds((n_seq // 2) * (row0 + j - NBUF), n_seq // 2)],
                    ssems[b]).wait()

            def sc_t(t, gb=gbufs[b], sb=sbufs[b]):
                tt = t // 2
                base = 64 * (t % 2)
                for q in range(4):
                    sb[tt, pl.ds(base + 16 * q, 16)] = (
                        gb[t, pl.ds(16 * q, 16)] * SCALE)

            plsc.parallel_loop(0, n_seq, unroll=4)(sc_t)

            @pl.when(j + NBUF < rows_w)
            def _issue(b=b, j=j):
                issue_gather(j + NBUF, b)

            pltpu.async_copy(
                sbufs[b],
                out_hbm.at[pl.ds((n_seq // 2) * (row0 + j), n_seq // 2)],
                ssems[b])
        return carry

    lax.fori_loop(0, rows_w // NBUF, outer_body, 0)

    for b in range(NBUF):
        j = rows_w - NBUF + b
        pltpu.make_async_copy(
            sbufs[b],
            out_hbm.at[pl.ds((n_seq // 2) * (row0 + j), n_seq // 2)],
            ssems[b]).wait()


@jax.jit
def _emb_call(x, lut):
    nb, ns = x.shape
    mesh = plsc.VectorSubcoreMesh(core_axis_name="c", subcore_axis_name="s")
    fn = functools.partial(
        pl.kernel,
        out_type=jax.ShapeDtypeStruct(
            (nb * ns * D_MODEL // 128, 128), jnp.float32),
        mesh=mesh,
        scratch_types=(
            [pltpu.VMEM((nb // NW, ns), jnp.int32)]
            + [pltpu.VMEM((ns, D_MODEL), jnp.float32)] * NBUF
            + [pltpu.VMEM((ns // 2, 128), jnp.float32)] * NBUF
            + [pltpu.SemaphoreType.DMA] * (2 * NBUF)
        ),
        compiler_params=pltpu.CompilerParams(
            use_tc_tiling_on_sc=False, needs_layout_passes=False),
    )(_emb_body)
    return fn(x, lut)


def kernel(x, lut):
    nb, ns = x.shape
    out = _emb_call(x.astype(jnp.int32), lut)
    return out.reshape(nb, ns, D_MODEL)
